# baseline (speedup 1.0000x reference)
import dataclasses

import jax
import jax.numpy as jnp
from jax import lax
from jax.experimental import pallas as pl
from jax.experimental.pallas import tpu as pltpu
from jax.experimental.pallas import tpu_sc as plsc

_B, _C, _H, _W = 128, 3, 256, 256
_T = 50
_TPAD = 64
_NW = 32
_ROWS_PER_B = _C * _H
_RB = _B // _NW
_CHUNK = 64
_NCH = _ROWS_PER_B // _CHUNK

_mesh = plsc.VectorSubcoreMesh(core_axis_name="c", subcore_axis_name="s")

_cp = pltpu.CompilerParams()
if "needs_layout_passes" in pltpu.CompilerParams.__dataclass_fields__:
    _cp = dataclasses.replace(_cp, needs_layout_passes=False)


def _sc_body(x_hbm, n_hbm, t_hbm, sac_hbm, som_hbm, o_hbm,
             xv, nv, ov, t_v, a_v, b_v, sem):
    wid = lax.axis_index("s") * 2 + lax.axis_index("c")

    pltpu.async_copy(t_hbm, t_v, sem).wait()
    pltpu.async_copy(sac_hbm, a_v, sem).wait()
    pltpu.async_copy(som_hbm, b_v, sem).wait()

    @pl.loop(0, _RB)
    def _(bi):
        bidx = wid * _RB + bi
        idx = jnp.full((16,), bidx, jnp.int32)
        tt = plsc.load_gather(t_v, [idx])
        av = plsc.load_gather(a_v, [tt])
        bv = plsc.load_gather(b_v, [tt])

        @pl.loop(0, _NCH)
        def _(ch):
            r0 = bidx * _ROWS_PER_B + ch * _CHUNK
            cx = pltpu.async_copy(x_hbm.at[pl.ds(r0, _CHUNK)], xv, sem)
            cn = pltpu.async_copy(n_hbm.at[pl.ds(r0, _CHUNK)], nv, sem)
            cx.wait()
            cn.wait()

            @pl.loop(0, _CHUNK)
            def _(r):
                @pl.loop(0, 256, step=64)
                def _(c0):
                    for u in range(4):
                        c = c0 + u * 16
                        ov[r, pl.ds(c, 16)] = (
                            av * xv[r, pl.ds(c, 16)] + bv * nv[r, pl.ds(c, 16)]
                        )

            pltpu.async_copy(ov, o_hbm.at[pl.ds(r0, _CHUNK)], sem).wait()


def kernel(x_start, t, noise, sqrt_alphas_cumprod, sqrt_one_minus_alphas_cumprod):
    x2 = x_start.reshape(_B * _ROWS_PER_B, 256)
    n2 = noise.reshape(_B * _ROWS_PER_B, 256)
    sac_p = jnp.pad(sqrt_alphas_cumprod, (0, _TPAD - _T))
    som_p = jnp.pad(sqrt_one_minus_alphas_cumprod, (0, _TPAD - _T))

    f = pl.kernel(
        _sc_body,
        out_type=jax.ShapeDtypeStruct((_B * _ROWS_PER_B, 256), jnp.float32),
        mesh=_mesh,
        scratch_types=[
            pltpu.VMEM((_CHUNK, 256), jnp.float32),
            pltpu.VMEM((_CHUNK, 256), jnp.float32),
            pltpu.VMEM((_CHUNK, 256), jnp.float32),
            pltpu.VMEM((_B,), jnp.int32),
            pltpu.VMEM((_TPAD,), jnp.float32),
            pltpu.VMEM((_TPAD,), jnp.float32),
            pltpu.SemaphoreType.DMA,
        ],
        compiler_params=_cp,
    )
    out = f(x2, n2, t, sac_p, som_p)
    return out.reshape(_B, _C, _H, _W)


# speedup vs baseline: 2.2384x; 2.2384x over previous
import jax
import jax.numpy as jnp
from jax.experimental import pallas as pl
from jax.experimental.pallas import tpu as pltpu

_B, _C, _H, _W = 128, 3, 256, 256
_CHW = _C * _H * _W
_LANES = 128
_SUB = _CHW // _LANES
_R = 4


def _qsample_body(t_ref, sac_ref, som_ref, x_ref, n_ref, o_ref):
    i = pl.program_id(0)
    for r in range(_R):
        tt = t_ref[i * _R + r]
        a = sac_ref[tt]
        b = som_ref[tt]
        o_ref[r] = a * x_ref[r] + b * n_ref[r]


def kernel(x_start, t, noise, sqrt_alphas_cumprod, sqrt_one_minus_alphas_cumprod):
    grid_spec = pltpu.PrefetchScalarGridSpec(
        num_scalar_prefetch=3,
        grid=(_B // _R,),
        in_specs=[
            pl.BlockSpec((_R, _C, _H, _W), lambda i, *_: (i, 0, 0, 0)),
            pl.BlockSpec((_R, _C, _H, _W), lambda i, *_: (i, 0, 0, 0)),
        ],
        out_specs=pl.BlockSpec((_R, _C, _H, _W), lambda i, *_: (i, 0, 0, 0)),
    )

    return pl.pallas_call(
        _qsample_body,
        grid_spec=grid_spec,
        out_shape=jax.ShapeDtypeStruct((_B, _C, _H, _W), jnp.float32),
    )(t, sqrt_alphas_cumprod, sqrt_one_minus_alphas_cumprod, x_start, noise)


# speedup vs baseline: 2.2388x; 1.0002x over previous
import jax
import jax.numpy as jnp
from jax.experimental import pallas as pl
from jax.experimental.pallas import tpu as pltpu

_B, _C, _H, _W = 128, 3, 256, 256
_CHW = _C * _H * _W
_LANES = 128
_SUB = _CHW // _LANES
_R = 8


def _qsample_body(t_ref, sac_ref, som_ref, x_ref, n_ref, o_ref):
    i = pl.program_id(0)
    for r in range(_R):
        tt = t_ref[i * _R + r]
        a = sac_ref[tt]
        b = som_ref[tt]
        o_ref[r] = a * x_ref[r] + b * n_ref[r]


def kernel(x_start, t, noise, sqrt_alphas_cumprod, sqrt_one_minus_alphas_cumprod):
    grid_spec = pltpu.PrefetchScalarGridSpec(
        num_scalar_prefetch=3,
        grid=(_B // _R,),
        in_specs=[
            pl.BlockSpec((_R, _C, _H, _W), lambda i, *_: (i, 0, 0, 0)),
            pl.BlockSpec((_R, _C, _H, _W), lambda i, *_: (i, 0, 0, 0)),
        ],
        out_specs=pl.BlockSpec((_R, _C, _H, _W), lambda i, *_: (i, 0, 0, 0)),
    )

    return pl.pallas_call(
        _qsample_body,
        grid_spec=grid_spec,
        out_shape=jax.ShapeDtypeStruct((_B, _C, _H, _W), jnp.float32),
    )(t, sqrt_alphas_cumprod, sqrt_one_minus_alphas_cumprod, x_start, noise)
